# baseline (speedup 1.0000x reference)
import functools

import jax
import jax.numpy as jnp
from jax import lax
from jax.experimental import pallas as pl
from jax.experimental.pallas import tpu as pltpu
from jax.experimental.pallas import tpu_sc as plsc

NC = 2
NS = 16
NW = NC * NS

CHUNK = 256
NBUF = 4


@functools.lru_cache(maxsize=None)
def _make_gather(B, V, D, b_per_w, n_chunks):
    mesh = plsc.VectorSubcoreMesh(core_axis_name="c", subcore_axis_name="s")
    nbuf = NBUF
    assert n_chunks % nbuf == 0 and n_chunks // nbuf >= 2
    n_blocks = n_chunks // nbuf
    row_bytes = CHUNK * D * 4

    @functools.partial(
        pl.kernel,
        out_type=jax.ShapeDtypeStruct((B, D), jnp.float32),
        mesh=mesh,
        compiler_params=pltpu.CompilerParams(use_tc_tiling_on_sc=False),
        scratch_types=[
            pltpu.VMEM((n_chunks, CHUNK), jnp.int32),
            [pltpu.VMEM((CHUNK, D), jnp.float32) for _ in range(nbuf)],
            pltpu.SemaphoreType.DMA,
        ],
    )
    def k(idx_hbm, table_hbm, out_hbm, idx_v, rows, gsem):
        wid = lax.axis_index("s") * NC + lax.axis_index("c")
        base = wid * b_per_w

        pltpu.sync_copy(idx_hbm.at[wid], idx_v)

        def start_gather(g, b):
            pltpu.async_copy(table_hbm.at[idx_v.at[g]], rows[b], gsem)

        def wait_gather(b):
            pltpu.make_async_copy(table_hbm.at[idx_v.at[0]], rows[b], gsem).wait()

        def copy_out(g, b):
            pltpu.sync_copy(rows[b], out_hbm.at[pl.ds(base + g * CHUNK, CHUNK)])

        for b in range(nbuf):
            start_gather(b, b)

        def body(blk, _):
            for b in range(nbuf):
                g = blk * nbuf + b
                wait_gather(b)
                copy_out(g, b)
                start_gather(g + nbuf, b)
            return _

        lax.fori_loop(0, n_blocks - 1, body, None)

        for b in range(nbuf):
            g = (n_blocks - 1) * nbuf + b
            wait_gather(b)
            copy_out(g, b)

    return k


def kernel(x, weight):
    BATCH, HIST = x.shape
    V, D = weight.shape
    B = BATCH * HIST
    b_per_w = B // NW
    n_chunks = b_per_w // CHUNK
    idx = x.reshape(NW, n_chunks, CHUNK).astype(jnp.int32)
    out = _make_gather(B, V, D, b_per_w, n_chunks)(idx, weight)
    return out.reshape(BATCH, HIST, D)


# speedup vs baseline: 1.0010x; 1.0010x over previous
import functools

import jax
import jax.numpy as jnp
from jax import lax
from jax.experimental import pallas as pl
from jax.experimental.pallas import tpu as pltpu
from jax.experimental.pallas import tpu_sc as plsc

NC = 2
NS = 16
NW = NC * NS

CHUNK = 256
NBUF = 4
G_AHEAD = 2
W_AHEAD = NBUF - G_AHEAD


@functools.lru_cache(maxsize=None)
def _make_gather(B, V, D, b_per_w, n_chunks):
    mesh = plsc.VectorSubcoreMesh(core_axis_name="c", subcore_axis_name="s")
    nbuf, G, W = NBUF, G_AHEAD, W_AHEAD
    assert n_chunks % nbuf == 0 and n_chunks // nbuf >= 3

    @functools.partial(
        pl.kernel,
        out_type=jax.ShapeDtypeStruct((B, D), jnp.float32),
        mesh=mesh,
        compiler_params=pltpu.CompilerParams(use_tc_tiling_on_sc=False),
        scratch_types=[
            pltpu.VMEM((n_chunks, CHUNK), jnp.int32),
            [pltpu.VMEM((CHUNK, D), jnp.float32) for _ in range(nbuf)],
            pltpu.SemaphoreType.DMA,
            pltpu.SemaphoreType.DMA,
        ],
    )
    def k(idx_hbm, table_hbm, out_hbm, idx_v, rows, gsem, osem):
        wid = lax.axis_index("s") * NC + lax.axis_index("c")
        base = wid * b_per_w

        pltpu.sync_copy(idx_hbm.at[wid], idx_v)

        def start_gather(g, b):
            pltpu.async_copy(table_hbm.at[idx_v.at[g]], rows[b], gsem)

        def wait_gather(b):
            pltpu.make_async_copy(table_hbm.at[idx_v.at[0]], rows[b], gsem).wait()

        def start_out(g, b):
            pltpu.async_copy(rows[b], out_hbm.at[pl.ds(base + g * CHUNK, CHUNK)],
                             osem)

        def wait_out(b):
            pltpu.make_async_copy(rows[b],
                                  out_hbm.at[pl.ds(base, CHUNK)], osem).wait()

        def step(g, b, do_owait, do_gissue):
            wait_gather(b)
            start_out(g, b)
            if do_owait:
                wait_out(b)
            if do_gissue:
                start_gather(g + G, (b + G) % nbuf)

        for g in range(G):
            start_gather(g, g % nbuf)

        for b in range(nbuf):
            step(b, b, b >= W, True)

        def body(blk, _):
            for b in range(nbuf):
                g = blk * nbuf + b
                step(g, b, True, True)
            return _

        lax.fori_loop(1, n_chunks // nbuf - 1, body, None,
                      unroll=False)

        for b in range(nbuf):
            g = n_chunks - nbuf + b
            step(g, b, g + G < n_chunks, g + G < n_chunks)

        for _ in range(min(W + G, n_chunks)):
            wait_out(0)

    return k


def kernel(x, weight):
    BATCH, HIST = x.shape
    V, D = weight.shape
    B = BATCH * HIST
    b_per_w = B // NW
    n_chunks = b_per_w // CHUNK
    idx = x.reshape(NW, n_chunks, CHUNK).astype(jnp.int32)
    out = _make_gather(B, V, D, b_per_w, n_chunks)(idx, weight)
    return out.reshape(BATCH, HIST, D)


# speedup vs baseline: 1.0518x; 1.0507x over previous
import functools

import jax
import jax.numpy as jnp
from jax import lax
from jax.experimental import pallas as pl
from jax.experimental.pallas import tpu as pltpu
from jax.experimental.pallas import tpu_sc as plsc

NC = 2
NS = 16
NW = NC * NS

CHUNK = 256
NBUF = 4
G_AHEAD = 2
W_AHEAD = NBUF - G_AHEAD


@functools.lru_cache(maxsize=None)
def _make_gather(B, V, D, b_per_w, n_chunks, BATCH_):
    mesh = plsc.VectorSubcoreMesh(core_axis_name="c", subcore_axis_name="s")
    nbuf, G, W = NBUF, G_AHEAD, W_AHEAD
    assert n_chunks % nbuf == 0 and n_chunks // nbuf >= 3

    @functools.partial(
        pl.kernel,
        out_type=jax.ShapeDtypeStruct((B // BATCH_, BATCH_, D), jnp.float32),
        mesh=mesh,
        compiler_params=pltpu.CompilerParams(use_tc_tiling_on_sc=False),
        scratch_types=[
            pltpu.VMEM((n_chunks, CHUNK), jnp.int32),
            [pltpu.VMEM((CHUNK, D), jnp.float32) for _ in range(nbuf)],
            pltpu.SemaphoreType.DMA,
            pltpu.SemaphoreType.DMA,
        ],
    )
    def k(idx_hbm, table_hbm, out_hbm, idx_v, rows, gsem, osem):
        wid = lax.axis_index("s") * NC + lax.axis_index("c")
        base = wid * b_per_w

        pltpu.sync_copy(idx_hbm.at[wid], idx_v)

        def start_gather(g, b):
            pltpu.async_copy(table_hbm.at[idx_v.at[g]], rows[b], gsem)

        def wait_gather(b):
            pltpu.make_async_copy(table_hbm.at[idx_v.at[0]], rows[b], gsem).wait()

        def start_out(g, b):
            flat = base + g * CHUNK
            pltpu.async_copy(
                rows[b],
                out_hbm.at[flat // BATCH_, pl.ds(lax.rem(flat, BATCH_), CHUNK)],
                osem)

        def wait_out(b):
            pltpu.make_async_copy(rows[b],
                                  out_hbm.at[0, pl.ds(0, CHUNK)],
                                  osem).wait()

        def step(g, b, do_owait, do_gissue):
            wait_gather(b)
            start_out(g, b)
            if do_owait:
                wait_out(b)
            if do_gissue:
                start_gather(g + G, (b + G) % nbuf)

        for g in range(G):
            start_gather(g, g % nbuf)

        for b in range(nbuf):
            step(b, b, b >= W, True)

        def body(blk, _):
            for b in range(nbuf):
                g = blk * nbuf + b
                step(g, b, True, True)
            return _

        lax.fori_loop(1, n_chunks // nbuf - 1, body, None,
                      unroll=False)

        for b in range(nbuf):
            g = n_chunks - nbuf + b
            step(g, b, g + G < n_chunks, g + G < n_chunks)

        for _ in range(min(W + G, n_chunks)):
            wait_out(0)

    return k


def kernel(x, weight):
    BATCH, HIST = x.shape
    V, D = weight.shape
    B = BATCH * HIST
    b_per_w = B // NW
    n_chunks = b_per_w // CHUNK
    idx = x.T.reshape(NW, n_chunks, CHUNK).astype(jnp.int32)
    out = _make_gather(B, V, D, b_per_w, n_chunks, BATCH)(idx, weight)
    return jnp.transpose(out, (1, 0, 2))
